# Initial kernel scaffold; baseline (speedup 1.0000x reference)
import jax
import jax.numpy as jnp
from jax.experimental import pallas as pl


def kernel(hidden_states, position_ids, W_qa, q_a_scale, W_qb, W_kva, kv_a_scale, W_kvb, W_o, W_q_idx, W_k_idx, w_idx):
    raise NotImplementedError("write your pallas kernel here")


# speedup vs baseline: 9.1984x; 9.1984x over previous
import functools
import numpy as np
import jax
import jax.numpy as jnp
from jax.experimental import pallas as pl
from jax.experimental.pallas import tpu as pltpu

S, HID = 2048, 2048
H, NOPE, ROPE, VDIM = 16, 128, 64, 128
QLORA, KVLORA = 1024, 512
IH, IDIM, TOPK = 4, 64, 512
EPS = 1e-5
SCALE = 1.0 / float(np.sqrt(NOPE + ROPE))
QB = 256
NEG = -1e30

_HI = jax.lax.Precision.HIGHEST


def _dot(a, b):
    return jnp.dot(a, b, preferred_element_type=jnp.float32, precision=_HI)


def _bdot(a, b):
    return jnp.dot(a.astype(jnp.bfloat16), b.astype(jnp.bfloat16),
                   preferred_element_type=jnp.float32)


def _bdot_nt(a, b):
    return jax.lax.dot_general(
        a.astype(jnp.bfloat16), b.astype(jnp.bfloat16),
        (((1,), (1,)), ((), ())), preferred_element_type=jnp.float32)


def _dot_nt(a, b):
    return jax.lax.dot_general(
        a, b, (((1,), (1,)), ((), ())),
        preferred_element_type=jnp.float32, precision=_HI)


def _proj_q(hs_ref, wqa_ref, qsc_ref, wqb_ref, wqi_ref, wki_ref,
            cos_ref, sin_ref, p_ref, q4_ref, qi4_ref, ki_ref):
    x = hs_ref[...]
    qa = _bdot(x, wqa_ref[...])
    var = jnp.mean(qa * qa, axis=1, keepdims=True)
    qn = qa * jax.lax.rsqrt(var + EPS) * qsc_ref[...]
    c, s = cos_ref[...], sin_ref[...]
    for h in range(H):
        qh = _bdot(qn, wqb_ref[h])
        qr = qh[:, NOPE:]
        qr = qr * c + _dot(qr, p_ref[...]) * s
        q4_ref[h, :, :NOPE] = qh[:, :NOPE]
        q4_ref[h, :, NOPE:] = qr
    for g in range(IH):
        qi4_ref[g] = _bdot(x, wqi_ref[g])
    ki_ref[...] = _bdot(x, wki_ref[...])


def _proj_kv(hs_ref, wkva_ref, ksc_ref, wkvb_ref, cos_ref, sin_ref, p_ref,
             k4_ref, v4_ref):
    x = hs_ref[...]
    kva = _bdot(x, wkva_ref[...])
    ckv = kva[:, :KVLORA]
    kr = kva[:, KVLORA:]
    var = jnp.mean(ckv * ckv, axis=1, keepdims=True)
    cn = ckv * jax.lax.rsqrt(var + EPS) * ksc_ref[...]
    c, s = cos_ref[...], sin_ref[...]
    kr = kr * c + _dot(kr, p_ref[...]) * s
    for h in range(H):
        kvh = _bdot(cn, wkvb_ref[h])
        k4_ref[h, :, :NOPE] = kvh[:, :NOPE]
        k4_ref[h, :, NOPE:] = kr
        v4_ref[h] = kvh[:, NOPE:]


def _select(qi_ref, ki_ref, w_ref, m_ref):
    ki = ki_ref[...]
    sc = jnp.zeros((QB, S), jnp.float32)
    for g in range(IH):
        r = jnp.maximum(_bdot_nt(qi_ref[g], ki), 0.0)
        sc = sc + w_ref[0, g] * r.astype(jnp.bfloat16).astype(jnp.float32)
    pid = pl.program_id(0)
    col = jax.lax.broadcasted_iota(jnp.int32, (QB, S), 1)
    row = pid * QB + jax.lax.broadcasted_iota(jnp.int32, (QB, S), 0)
    causal = col <= row
    key = jnp.where(causal, jax.lax.bitcast_convert_type(sc, jnp.int32),
                    jnp.int32(-1))
    lo = jnp.full((QB, 1), -1, jnp.int32)
    hi = jnp.max(key, axis=1, keepdims=True) + 1
    for _ in range(32):
        mid = lo + jax.lax.shift_right_logical(hi - lo, 1)
        cnt = jnp.sum((key >= mid).astype(jnp.int32), axis=1, keepdims=True)
        ok = cnt >= TOPK
        lo = jnp.where(ok, mid, lo)
        hi = jnp.where(ok, hi, mid)
    t = lo
    cgt = jnp.sum((key > t).astype(jnp.int32), axis=1, keepdims=True)
    need = TOPK - cgt
    eq = key == t
    tlo = jnp.zeros((QB, 1), jnp.int32)
    thi = jnp.full((QB, 1), S, jnp.int32)
    for _ in range(11):
        mid = tlo + jax.lax.shift_right_logical(thi - tlo, 1)
        cnt = jnp.sum((eq & (col < mid)).astype(jnp.int32),
                      axis=1, keepdims=True)
        ok = cnt >= need
        thi = jnp.where(ok, mid, thi)
        tlo = jnp.where(ok, tlo, mid)
    sel = ((key > t) | (eq & (col < thi))) & causal
    m_ref[...] = sel.astype(jnp.int8)


def _attn(q_ref, k_ref, v_ref, m_ref, o_ref):
    sc = _bdot_nt(q_ref[0], k_ref[0])
    sc = jnp.where(m_ref[...] != 0, sc * SCALE, NEG)
    mx = jnp.max(sc, axis=1, keepdims=True)
    p = jnp.exp(sc - mx)
    den = jnp.sum(p, axis=1, keepdims=True)
    o_ref[0] = _bdot(p / den, v_ref[0])


def _outproj(a_ref, wo_ref, o_ref):
    h = pl.program_id(1)
    part = _bdot(a_ref[0], wo_ref[0])

    @pl.when(h == 0)
    def _():
        o_ref[...] = part

    @pl.when(h > 0)
    def _():
        o_ref[...] = o_ref[...] + part


def _full(shape):
    n = len(shape)
    return pl.BlockSpec(shape, lambda *_: (0,) * n)


@jax.jit
def kernel(hidden_states, position_ids, W_qa, q_a_scale, W_qb, W_kva,
           kv_a_scale, W_kvb, W_o, W_q_idx, W_k_idx, w_idx):
    hs = hidden_states.reshape(S, HID)
    wqb4 = W_qb.reshape(QLORA, H, NOPE + ROPE).transpose(1, 0, 2)
    wkvb4 = W_kvb.reshape(KVLORA, H, NOPE + VDIM).transpose(1, 0, 2)
    wqi4 = W_q_idx.reshape(HID, IH, IDIM).transpose(1, 0, 2)
    wo3 = W_o.reshape(H, VDIM, HID)
    qsc = q_a_scale.reshape(1, QLORA)
    ksc = kv_a_scale.reshape(1, KVLORA)
    w2 = w_idx.reshape(1, IH)
    inv = 1.0 / (10000.0 ** (np.arange(0, ROPE, 2, dtype=np.float32) / ROPE))
    inv2 = jnp.asarray(np.repeat(inv, 2).reshape(1, ROPE))
    fr = position_ids.reshape(S, 1).astype(jnp.float32) * inv2
    cos_t = jnp.cos(fr)
    sin_t = jnp.sin(fr)
    pmat = np.zeros((ROPE, ROPE), np.float32)
    for i in range(0, ROPE, 2):
        pmat[i + 1, i] = -1.0
        pmat[i, i + 1] = 1.0
    pmat = jnp.asarray(pmat)

    nq = S // QB
    f32 = jnp.float32

    q4, qi4, ki = pl.pallas_call(
        _proj_q,
        grid=(nq,),
        in_specs=[
            pl.BlockSpec((QB, HID), lambda i: (i, 0)),
            _full((HID, QLORA)),
            _full((1, QLORA)),
            _full((H, QLORA, NOPE + ROPE)),
            _full((IH, HID, IDIM)),
            _full((HID, IDIM)),
            pl.BlockSpec((QB, ROPE), lambda i: (i, 0)),
            pl.BlockSpec((QB, ROPE), lambda i: (i, 0)),
            _full((ROPE, ROPE)),
        ],
        out_specs=[
            pl.BlockSpec((H, QB, NOPE + ROPE), lambda i: (0, i, 0)),
            pl.BlockSpec((IH, QB, IDIM), lambda i: (0, i, 0)),
            pl.BlockSpec((QB, IDIM), lambda i: (i, 0)),
        ],
        out_shape=[
            jax.ShapeDtypeStruct((H, S, NOPE + ROPE), f32),
            jax.ShapeDtypeStruct((IH, S, IDIM), f32),
            jax.ShapeDtypeStruct((S, IDIM), f32),
        ],
    )(hs, W_qa, qsc, wqb4, wqi4, W_k_idx, cos_t, sin_t, pmat)

    k4, v4 = pl.pallas_call(
        _proj_kv,
        grid=(nq,),
        in_specs=[
            pl.BlockSpec((QB, HID), lambda i: (i, 0)),
            _full((HID, KVLORA + ROPE)),
            _full((1, KVLORA)),
            _full((H, KVLORA, NOPE + VDIM)),
            pl.BlockSpec((QB, ROPE), lambda i: (i, 0)),
            pl.BlockSpec((QB, ROPE), lambda i: (i, 0)),
            _full((ROPE, ROPE)),
        ],
        out_specs=[
            pl.BlockSpec((H, QB, NOPE + ROPE), lambda i: (0, i, 0)),
            pl.BlockSpec((H, QB, VDIM), lambda i: (0, i, 0)),
        ],
        out_shape=[
            jax.ShapeDtypeStruct((H, S, NOPE + ROPE), f32),
            jax.ShapeDtypeStruct((H, S, VDIM), f32),
        ],
    )(hs, W_kva, ksc, wkvb4, cos_t, sin_t, pmat)

    mask = pl.pallas_call(
        _select,
        grid=(nq,),
        in_specs=[
            pl.BlockSpec((IH, QB, IDIM), lambda i: (0, i, 0)),
            _full((S, IDIM)),
            _full((1, IH)),
        ],
        out_specs=pl.BlockSpec((QB, S), lambda i: (i, 0)),
        out_shape=jax.ShapeDtypeStruct((S, S), jnp.int8),
    )(qi4, ki, w2)

    att4 = pl.pallas_call(
        _attn,
        grid=(H, nq),
        in_specs=[
            pl.BlockSpec((1, QB, NOPE + ROPE), lambda h, i: (h, i, 0)),
            pl.BlockSpec((1, S, NOPE + ROPE), lambda h, i: (h, 0, 0)),
            pl.BlockSpec((1, S, VDIM), lambda h, i: (h, 0, 0)),
            pl.BlockSpec((QB, S), lambda h, i: (i, 0)),
        ],
        out_specs=pl.BlockSpec((1, QB, VDIM), lambda h, i: (h, i, 0)),
        out_shape=jax.ShapeDtypeStruct((H, S, VDIM), f32),
    )(q4, k4, v4, mask)

    out = pl.pallas_call(
        _outproj,
        grid=(nq, H),
        in_specs=[
            pl.BlockSpec((1, QB, VDIM), lambda i, h: (h, i, 0)),
            pl.BlockSpec((1, VDIM, HID), lambda i, h: (h, 0, 0)),
        ],
        out_specs=pl.BlockSpec((QB, HID), lambda i, h: (i, 0)),
        out_shape=jax.ShapeDtypeStruct((S, HID), f32),
        compiler_params=pltpu.CompilerParams(
            dimension_semantics=("parallel", "arbitrary")),
    )(att4, wo3)

    return out.reshape(1, S, HID)


# speedup vs baseline: 9.6609x; 1.0503x over previous
import functools
import numpy as np
import jax
import jax.numpy as jnp
from jax.experimental import pallas as pl
from jax.experimental.pallas import tpu as pltpu

S, HID = 2048, 2048
H, NOPE, ROPE, VDIM = 16, 128, 64, 128
QLORA, KVLORA = 1024, 512
IH, IDIM, TOPK = 4, 64, 512
EPS = 1e-5
SCALE = 1.0 / float(np.sqrt(NOPE + ROPE))
QB = 256
NEG = -1e30

_HI = jax.lax.Precision.HIGHEST


def _dot(a, b):
    return jnp.dot(a, b, preferred_element_type=jnp.float32, precision=_HI)


def _f32dot(a, b):
    return jnp.dot(a, b, preferred_element_type=jnp.float32)


def _f32dot_nt(a, b):
    return jax.lax.dot_general(
        a, b, (((1,), (1,)), ((), ())), preferred_element_type=jnp.float32)


def _dot_nt(a, b):
    return jax.lax.dot_general(
        a, b, (((1,), (1,)), ((), ())),
        preferred_element_type=jnp.float32, precision=_HI)


def _proj_q(hs_ref, wqa_ref, qsc_ref, wqb_ref, wqi_ref, wki_ref,
            cos_ref, sin_ref, p_ref, q4_ref, qi4_ref, ki_ref):
    xb = hs_ref[...].astype(jnp.bfloat16)
    qa = _f32dot(xb, wqa_ref[...])
    var = jnp.mean(qa * qa, axis=1, keepdims=True)
    qn = qa * jax.lax.rsqrt(var + EPS) * qsc_ref[...]
    qnb = qn.astype(jnp.bfloat16)
    c, s = cos_ref[...], sin_ref[...]
    for h in range(H):
        qh = _f32dot(qnb, wqb_ref[h])
        qr = qh[:, NOPE:]
        qr = qr * c + _dot(qr, p_ref[...]) * s
        q4_ref[h, :, :NOPE] = qh[:, :NOPE].astype(jnp.bfloat16)
        q4_ref[h, :, NOPE:] = qr.astype(jnp.bfloat16)
    for g in range(IH):
        qi4_ref[g] = _f32dot(xb, wqi_ref[g]).astype(jnp.bfloat16)
    ki_ref[...] = _f32dot(xb, wki_ref[...]).astype(jnp.bfloat16)


def _proj_kv(hs_ref, wkva_ref, ksc_ref, wkvb_ref, cos_ref, sin_ref, p_ref,
             k4_ref, v4_ref):
    xb = hs_ref[...].astype(jnp.bfloat16)
    kva = _f32dot(xb, wkva_ref[...])
    ckv = kva[:, :KVLORA]
    kr = kva[:, KVLORA:]
    var = jnp.mean(ckv * ckv, axis=1, keepdims=True)
    cn = ckv * jax.lax.rsqrt(var + EPS) * ksc_ref[...]
    cnb = cn.astype(jnp.bfloat16)
    c, s = cos_ref[...], sin_ref[...]
    kr = kr * c + _dot(kr, p_ref[...]) * s
    krb = kr.astype(jnp.bfloat16)
    for h in range(H):
        kvh = _f32dot(cnb, wkvb_ref[h])
        k4_ref[h, :, :NOPE] = kvh[:, :NOPE].astype(jnp.bfloat16)
        k4_ref[h, :, NOPE:] = krb
        v4_ref[h] = kvh[:, NOPE:].astype(jnp.bfloat16)


def _select(qi_ref, ki_ref, w_ref, m_ref):
    ki = ki_ref[...]
    sc = jnp.zeros((QB, S), jnp.float32)
    for g in range(IH):
        r = jnp.maximum(_f32dot_nt(qi_ref[g], ki), 0.0)
        sc = sc + w_ref[0, g] * r.astype(jnp.bfloat16).astype(jnp.float32)
    pid = pl.program_id(0)
    col = jax.lax.broadcasted_iota(jnp.int32, (QB, S), 1)
    row = pid * QB + jax.lax.broadcasted_iota(jnp.int32, (QB, S), 0)
    causal = col <= row
    key = jnp.where(causal, jax.lax.bitcast_convert_type(sc, jnp.int32),
                    jnp.int32(-1))
    lo = jnp.full((QB, 1), -1, jnp.int32)
    hi = jnp.max(key, axis=1, keepdims=True) + 1
    for _ in range(32):
        mid = lo + jax.lax.shift_right_logical(hi - lo, 1)
        cnt = jnp.sum((key >= mid).astype(jnp.int32), axis=1, keepdims=True)
        ok = cnt >= TOPK
        lo = jnp.where(ok, mid, lo)
        hi = jnp.where(ok, hi, mid)
    t = lo
    cgt = jnp.sum((key > t).astype(jnp.int32), axis=1, keepdims=True)
    need = TOPK - cgt
    eq = key == t
    tlo = jnp.zeros((QB, 1), jnp.int32)
    thi = jnp.full((QB, 1), S, jnp.int32)
    for _ in range(11):
        mid = tlo + jax.lax.shift_right_logical(thi - tlo, 1)
        cnt = jnp.sum((eq & (col < mid)).astype(jnp.int32),
                      axis=1, keepdims=True)
        ok = cnt >= need
        thi = jnp.where(ok, mid, thi)
        tlo = jnp.where(ok, tlo, mid)
    sel = ((key > t) | (eq & (col < thi))) & causal
    m_ref[...] = sel.astype(jnp.int8)


def _attn(q_ref, k_ref, v_ref, m_ref, o_ref):
    sc = _f32dot_nt(q_ref[0], k_ref[0])
    sc = jnp.where(m_ref[...] != 0, sc * SCALE, NEG)
    mx = jnp.max(sc, axis=1, keepdims=True)
    p = jnp.exp(sc - mx)
    den = jnp.sum(p, axis=1, keepdims=True)
    pb = (p / den).astype(jnp.bfloat16)
    o_ref[0] = _f32dot(pb, v_ref[0]).astype(jnp.bfloat16)


def _outproj(a_ref, wo_ref, o_ref):
    h = pl.program_id(1)
    part = _f32dot(a_ref[0], wo_ref[0])

    @pl.when(h == 0)
    def _():
        o_ref[...] = part

    @pl.when(h > 0)
    def _():
        o_ref[...] = o_ref[...] + part


def _full(shape):
    n = len(shape)
    return pl.BlockSpec(shape, lambda *_: (0,) * n)


@jax.jit
def kernel(hidden_states, position_ids, W_qa, q_a_scale, W_qb, W_kva,
           kv_a_scale, W_kvb, W_o, W_q_idx, W_k_idx, w_idx):
    hs = hidden_states.reshape(S, HID)
    bf16 = jnp.bfloat16
    wqa_b = W_qa.astype(bf16)
    wkva_b = W_kva.astype(bf16)
    wki_b = W_k_idx.astype(bf16)
    wqb4 = W_qb.astype(bf16).reshape(QLORA, H, NOPE + ROPE).transpose(1, 0, 2)
    wkvb4 = W_kvb.astype(bf16).reshape(KVLORA, H, NOPE + VDIM).transpose(1, 0, 2)
    wqi4 = W_q_idx.astype(bf16).reshape(HID, IH, IDIM).transpose(1, 0, 2)
    wo3 = W_o.astype(bf16).reshape(H, VDIM, HID)
    qsc = q_a_scale.reshape(1, QLORA)
    ksc = kv_a_scale.reshape(1, KVLORA)
    w2 = w_idx.reshape(1, IH)
    inv = 1.0 / (10000.0 ** (np.arange(0, ROPE, 2, dtype=np.float32) / ROPE))
    inv2 = jnp.asarray(np.repeat(inv, 2).reshape(1, ROPE))
    fr = position_ids.reshape(S, 1).astype(jnp.float32) * inv2
    cos_t = jnp.cos(fr)
    sin_t = jnp.sin(fr)
    pmat = np.zeros((ROPE, ROPE), np.float32)
    for i in range(0, ROPE, 2):
        pmat[i + 1, i] = -1.0
        pmat[i, i + 1] = 1.0
    pmat = jnp.asarray(pmat)

    nq = S // QB
    f32 = jnp.float32

    q4, qi4, ki = pl.pallas_call(
        _proj_q,
        grid=(nq,),
        in_specs=[
            pl.BlockSpec((QB, HID), lambda i: (i, 0)),
            _full((HID, QLORA)),
            _full((1, QLORA)),
            _full((H, QLORA, NOPE + ROPE)),
            _full((IH, HID, IDIM)),
            _full((HID, IDIM)),
            pl.BlockSpec((QB, ROPE), lambda i: (i, 0)),
            pl.BlockSpec((QB, ROPE), lambda i: (i, 0)),
            _full((ROPE, ROPE)),
        ],
        out_specs=[
            pl.BlockSpec((H, QB, NOPE + ROPE), lambda i: (0, i, 0)),
            pl.BlockSpec((IH, QB, IDIM), lambda i: (0, i, 0)),
            pl.BlockSpec((QB, IDIM), lambda i: (i, 0)),
        ],
        out_shape=[
            jax.ShapeDtypeStruct((H, S, NOPE + ROPE), jnp.bfloat16),
            jax.ShapeDtypeStruct((IH, S, IDIM), jnp.bfloat16),
            jax.ShapeDtypeStruct((S, IDIM), jnp.bfloat16),
        ],
    )(hs, wqa_b, qsc, wqb4, wqi4, wki_b, cos_t, sin_t, pmat)

    k4, v4 = pl.pallas_call(
        _proj_kv,
        grid=(nq,),
        in_specs=[
            pl.BlockSpec((QB, HID), lambda i: (i, 0)),
            _full((HID, KVLORA + ROPE)),
            _full((1, KVLORA)),
            _full((H, KVLORA, NOPE + VDIM)),
            pl.BlockSpec((QB, ROPE), lambda i: (i, 0)),
            pl.BlockSpec((QB, ROPE), lambda i: (i, 0)),
            _full((ROPE, ROPE)),
        ],
        out_specs=[
            pl.BlockSpec((H, QB, NOPE + ROPE), lambda i: (0, i, 0)),
            pl.BlockSpec((H, QB, VDIM), lambda i: (0, i, 0)),
        ],
        out_shape=[
            jax.ShapeDtypeStruct((H, S, NOPE + ROPE), jnp.bfloat16),
            jax.ShapeDtypeStruct((H, S, VDIM), jnp.bfloat16),
        ],
    )(hs, wkva_b, ksc, wkvb4, cos_t, sin_t, pmat)

    mask = pl.pallas_call(
        _select,
        grid=(nq,),
        in_specs=[
            pl.BlockSpec((IH, QB, IDIM), lambda i: (0, i, 0)),
            _full((S, IDIM)),
            _full((1, IH)),
        ],
        out_specs=pl.BlockSpec((QB, S), lambda i: (i, 0)),
        out_shape=jax.ShapeDtypeStruct((S, S), jnp.int8),
    )(qi4, ki, w2)

    att4 = pl.pallas_call(
        _attn,
        grid=(H, nq),
        in_specs=[
            pl.BlockSpec((1, QB, NOPE + ROPE), lambda h, i: (h, i, 0)),
            pl.BlockSpec((1, S, NOPE + ROPE), lambda h, i: (h, 0, 0)),
            pl.BlockSpec((1, S, VDIM), lambda h, i: (h, 0, 0)),
            pl.BlockSpec((QB, S), lambda h, i: (i, 0)),
        ],
        out_specs=pl.BlockSpec((1, QB, VDIM), lambda h, i: (h, i, 0)),
        out_shape=jax.ShapeDtypeStruct((H, S, VDIM), jnp.bfloat16),
    )(q4, k4, v4, mask)

    out = pl.pallas_call(
        _outproj,
        grid=(nq, H),
        in_specs=[
            pl.BlockSpec((1, QB, VDIM), lambda i, h: (h, i, 0)),
            pl.BlockSpec((1, VDIM, HID), lambda i, h: (h, 0, 0)),
        ],
        out_specs=pl.BlockSpec((QB, HID), lambda i, h: (i, 0)),
        out_shape=jax.ShapeDtypeStruct((S, HID), f32),
        compiler_params=pltpu.CompilerParams(
            dimension_semantics=("parallel", "arbitrary")),
    )(att4, wo3)

    return out.reshape(1, S, HID)


# speedup vs baseline: 11.8588x; 1.2275x over previous
import numpy as np
import jax
import jax.numpy as jnp
from jax.experimental import pallas as pl
from jax.experimental.pallas import tpu as pltpu

S, HID = 2048, 2048
H, NOPE, ROPE, VDIM = 16, 128, 64, 128
QLORA, KVLORA = 1024, 512
IH, IDIM, TOPK = 4, 64, 512
EPS = 1e-5
SCALE = 1.0 / float(np.sqrt(NOPE + ROPE))
QB = 256
NEG = -1e30


def _dot(a, b):
    return jnp.dot(a, b, preferred_element_type=jnp.float32,
                   precision=jax.lax.Precision.HIGHEST)


def _f32dot(a, b):
    return jnp.dot(a, b, preferred_element_type=jnp.float32)


def _f32dot_nt(a, b):
    return jax.lax.dot_general(
        a, b, (((1,), (1,)), ((), ())), preferred_element_type=jnp.float32)


def _proj_q(hs_ref, wqa_ref, qsc_ref, wqb_ref, wqi_ref, wki_ref,
            cos_ref, sin_ref, p_ref, q4_ref, qi4_ref, ki_ref):
    xb = hs_ref[...].astype(jnp.bfloat16)
    qa = _f32dot(xb, wqa_ref[...])
    var = jnp.mean(qa * qa, axis=1, keepdims=True)
    qn = qa * jax.lax.rsqrt(var + EPS) * qsc_ref[...]
    qnb = qn.astype(jnp.bfloat16)
    c, s = cos_ref[...], sin_ref[...]
    for h in range(H):
        qh = _f32dot(qnb, wqb_ref[h])
        qr = qh[:, NOPE:]
        qr = qr * c + _dot(qr, p_ref[...]) * s
        q4_ref[h, :, :NOPE] = qh[:, :NOPE].astype(jnp.bfloat16)
        q4_ref[h, :, NOPE:] = qr.astype(jnp.bfloat16)
    for g in range(IH):
        qi4_ref[g] = _f32dot(xb, wqi_ref[g]).astype(jnp.bfloat16)
    ki_ref[...] = _f32dot(xb, wki_ref[...]).astype(jnp.bfloat16)


def _proj_kv(hs_ref, wkva_ref, ksc_ref, wkvb_ref, cos_ref, sin_ref, p_ref,
             k4_ref, v4_ref):
    xb = hs_ref[...].astype(jnp.bfloat16)
    kva = _f32dot(xb, wkva_ref[...])
    ckv = kva[:, :KVLORA]
    kr = kva[:, KVLORA:]
    var = jnp.mean(ckv * ckv, axis=1, keepdims=True)
    cn = ckv * jax.lax.rsqrt(var + EPS) * ksc_ref[...]
    cnb = cn.astype(jnp.bfloat16)
    c, s = cos_ref[...], sin_ref[...]
    kr = kr * c + _dot(kr, p_ref[...]) * s
    krb = kr.astype(jnp.bfloat16)
    for h in range(H):
        kvh = _f32dot(cnb, wkvb_ref[h])
        k4_ref[h, :, :NOPE] = kvh[:, :NOPE].astype(jnp.bfloat16)
        k4_ref[h, :, NOPE:] = krb
        v4_ref[h] = kvh[:, NOPE:].astype(jnp.bfloat16)


def _make_select(width, c0, tie_iters):
    def _select(qi_ref, ki_ref, w_ref, m_ref):
        ki = ki_ref[...]
        sc = jnp.zeros((QB, width), jnp.float32)
        for g in range(IH):
            r = jnp.maximum(_f32dot_nt(qi_ref[g], ki), 0.0)
            sc = sc + w_ref[0, g] * r.astype(jnp.bfloat16).astype(jnp.float32)
        pid = pl.program_id(0)
        col = jax.lax.broadcasted_iota(jnp.int32, (QB, width), 1)
        row = (c0 + pid) * QB + jax.lax.broadcasted_iota(
            jnp.int32, (QB, width), 0)
        causal = col <= row
        key = jnp.where(causal, jax.lax.bitcast_convert_type(sc, jnp.int32),
                        jnp.int32(-1))
        lo = jnp.full((QB, 1), -1, jnp.int32)
        hi = jnp.max(key, axis=1, keepdims=True) + 1
        for _ in range(32):
            mid = lo + jax.lax.shift_right_logical(hi - lo, 1)
            cnt = jnp.sum((key >= mid).astype(jnp.int32), axis=1,
                          keepdims=True)
            ok = cnt >= TOPK
            lo = jnp.where(ok, mid, lo)
            hi = jnp.where(ok, hi, mid)
        t = lo
        cgt = jnp.sum((key > t).astype(jnp.int32), axis=1, keepdims=True)
        need = TOPK - cgt
        eq = key == t
        tlo = jnp.zeros((QB, 1), jnp.int32)
        thi = jnp.full((QB, 1), width, jnp.int32)
        for _ in range(tie_iters):
            mid = tlo + jax.lax.shift_right_logical(thi - tlo, 1)
            cnt = jnp.sum((eq & (col < mid)).astype(jnp.int32),
                          axis=1, keepdims=True)
            ok = cnt >= need
            thi = jnp.where(ok, mid, thi)
            tlo = jnp.where(ok, tlo, mid)
        sel = ((key > t) | (eq & (col < thi))) & causal
        m_ref[:, :width] = sel.astype(jnp.int8)
        if width < S:
            m_ref[:, width:] = jnp.zeros((QB, S - width), jnp.int8)
    return _select


def _attn(q_ref, k_ref, v_ref, m_ref, o_ref):
    sc = _f32dot_nt(q_ref[0], k_ref[0])
    sc = jnp.where(m_ref[...] != 0, sc * SCALE, NEG)
    mx = jnp.max(sc, axis=1, keepdims=True)
    p = jnp.exp(sc - mx)
    den = jnp.sum(p, axis=1, keepdims=True)
    pb = (p / den).astype(jnp.bfloat16)
    o_ref[...] = _f32dot(pb, v_ref[0]).astype(jnp.bfloat16)


def _outproj(a_ref, wo_ref, o_ref):
    o_ref[...] = _f32dot(a_ref[...], wo_ref[...])


def _full(shape):
    n = len(shape)
    return pl.BlockSpec(shape, lambda *_: (0,) * n)


@jax.jit
def kernel(hidden_states, position_ids, W_qa, q_a_scale, W_qb, W_kva,
           kv_a_scale, W_kvb, W_o, W_q_idx, W_k_idx, w_idx):
    hs = hidden_states.reshape(S, HID)
    bf16 = jnp.bfloat16
    wqa_b = W_qa.astype(bf16)
    wkva_b = W_kva.astype(bf16)
    wki_b = W_k_idx.astype(bf16)
    wo_b = W_o.astype(bf16)
    wqb4 = W_qb.astype(bf16).reshape(QLORA, H, NOPE + ROPE).transpose(1, 0, 2)
    wkvb4 = W_kvb.astype(bf16).reshape(KVLORA, H, NOPE + VDIM).transpose(1, 0, 2)
    wqi4 = W_q_idx.astype(bf16).reshape(HID, IH, IDIM).transpose(1, 0, 2)
    qsc = q_a_scale.reshape(1, QLORA)
    ksc = kv_a_scale.reshape(1, KVLORA)
    w2 = w_idx.reshape(1, IH)
    inv = 1.0 / (10000.0 ** (np.arange(0, ROPE, 2, dtype=np.float32) / ROPE))
    inv2 = jnp.asarray(np.repeat(inv, 2).reshape(1, ROPE))
    fr = position_ids.reshape(S, 1).astype(jnp.float32) * inv2
    cos_t = jnp.cos(fr)
    sin_t = jnp.sin(fr)
    pmat = np.zeros((ROPE, ROPE), np.float32)
    for i in range(0, ROPE, 2):
        pmat[i + 1, i] = -1.0
        pmat[i, i + 1] = 1.0
    pmat = jnp.asarray(pmat)

    nq = S // QB
    f32 = jnp.float32

    q4, qi4, ki = pl.pallas_call(
        _proj_q,
        grid=(nq,),
        in_specs=[
            pl.BlockSpec((QB, HID), lambda i: (i, 0)),
            _full((HID, QLORA)),
            _full((1, QLORA)),
            _full((H, QLORA, NOPE + ROPE)),
            _full((IH, HID, IDIM)),
            _full((HID, IDIM)),
            pl.BlockSpec((QB, ROPE), lambda i: (i, 0)),
            pl.BlockSpec((QB, ROPE), lambda i: (i, 0)),
            _full((ROPE, ROPE)),
        ],
        out_specs=[
            pl.BlockSpec((H, QB, NOPE + ROPE), lambda i: (0, i, 0)),
            pl.BlockSpec((IH, QB, IDIM), lambda i: (0, i, 0)),
            pl.BlockSpec((QB, IDIM), lambda i: (i, 0)),
        ],
        out_shape=[
            jax.ShapeDtypeStruct((H, S, NOPE + ROPE), bf16),
            jax.ShapeDtypeStruct((IH, S, IDIM), bf16),
            jax.ShapeDtypeStruct((S, IDIM), bf16),
        ],
    )(hs, wqa_b, qsc, wqb4, wqi4, wki_b, cos_t, sin_t, pmat)

    k4, v4 = pl.pallas_call(
        _proj_kv,
        grid=(nq,),
        in_specs=[
            pl.BlockSpec((QB, HID), lambda i: (i, 0)),
            _full((HID, KVLORA + ROPE)),
            _full((1, KVLORA)),
            _full((H, KVLORA, NOPE + VDIM)),
            pl.BlockSpec((QB, ROPE), lambda i: (i, 0)),
            pl.BlockSpec((QB, ROPE), lambda i: (i, 0)),
            _full((ROPE, ROPE)),
        ],
        out_specs=[
            pl.BlockSpec((H, QB, NOPE + ROPE), lambda i: (0, i, 0)),
            pl.BlockSpec((H, QB, VDIM), lambda i: (0, i, 0)),
        ],
        out_shape=[
            jax.ShapeDtypeStruct((H, S, NOPE + ROPE), bf16),
            jax.ShapeDtypeStruct((H, S, VDIM), bf16),
        ],
    )(hs, wkva_b, ksc, wkvb4, cos_t, sin_t, pmat)

    masks = []
    for ci in range(4):
        width = (ci + 1) * 2 * QB
        tie_iters = max(1, int(np.ceil(np.log2(width))))
        masks.append(pl.pallas_call(
            _make_select(width, ci * 2, tie_iters),
            grid=(2,),
            in_specs=[
                pl.BlockSpec((IH, QB, IDIM),
                             lambda i, c=ci: (0, c * 2 + i, 0)),
                pl.BlockSpec((width, IDIM), lambda i: (0, 0)),
                _full((1, IH)),
            ],
            out_specs=pl.BlockSpec((QB, S), lambda i: (i, 0)),
            out_shape=jax.ShapeDtypeStruct((2 * QB, S), jnp.int8),
        )(qi4, ki, w2))
    mask = jnp.concatenate(masks, axis=0)

    att2 = pl.pallas_call(
        _attn,
        grid=(H, nq),
        in_specs=[
            pl.BlockSpec((1, QB, NOPE + ROPE), lambda h, i: (h, i, 0)),
            pl.BlockSpec((1, S, NOPE + ROPE), lambda h, i: (h, 0, 0)),
            pl.BlockSpec((1, S, VDIM), lambda h, i: (h, 0, 0)),
            pl.BlockSpec((QB, S), lambda h, i: (i, 0)),
        ],
        out_specs=pl.BlockSpec((QB, VDIM), lambda h, i: (i, h)),
        out_shape=jax.ShapeDtypeStruct((S, H * VDIM), bf16),
    )(q4, k4, v4, mask)

    out = pl.pallas_call(
        _outproj,
        grid=(nq,),
        in_specs=[
            pl.BlockSpec((QB, H * VDIM), lambda i: (i, 0)),
            _full((H * VDIM, HID)),
        ],
        out_specs=pl.BlockSpec((QB, HID), lambda i: (i, 0)),
        out_shape=jax.ShapeDtypeStruct((S, HID), f32),
    )(att2, wo_b)

    return out.reshape(1, S, HID)


# speedup vs baseline: 13.5280x; 1.1408x over previous
import numpy as np
import jax
import jax.numpy as jnp
from jax.experimental import pallas as pl
from jax.experimental.pallas import tpu as pltpu

S, HID = 2048, 2048
H, NOPE, ROPE, VDIM = 16, 128, 64, 128
QLORA, KVLORA = 1024, 512
IH, IDIM, TOPK = 4, 64, 512
EPS = 1e-5
SCALE = 1.0 / float(np.sqrt(NOPE + ROPE))
QB = 256
NEG = -1e30


def _pairswap(x):
    even = jax.lax.broadcasted_iota(jnp.int32, x.shape, 1) % 2 == 0
    n = x.shape[1]
    return jnp.where(even, -pltpu.roll(x, n - 1, 1), pltpu.roll(x, 1, 1))


def _f32dot(a, b):
    return jnp.dot(a, b, preferred_element_type=jnp.float32)


def _f32dot_nt(a, b):
    return jax.lax.dot_general(
        a, b, (((1,), (1,)), ((), ())), preferred_element_type=jnp.float32)


def _proj_q(hs_ref, wqa_ref, qsc_ref, wqb_ref, wqi_ref, wki_ref,
            cos_ref, sin_ref, q4_ref, qi4_ref, ki_ref):
    xb = hs_ref[...].astype(jnp.bfloat16)
    qa = _f32dot(xb, wqa_ref[...])
    var = jnp.mean(qa * qa, axis=1, keepdims=True)
    qn = qa * jax.lax.rsqrt(var + EPS) * qsc_ref[...]
    qnb = qn.astype(jnp.bfloat16)
    c, s = cos_ref[...], sin_ref[...]
    for h in range(H):
        qh = _f32dot(qnb, wqb_ref[h])
        qr = qh[:, NOPE:]
        qr = qr * c + _pairswap(qr) * s
        q4_ref[h, :, :NOPE] = qh[:, :NOPE].astype(jnp.bfloat16)
        q4_ref[h, :, NOPE:] = qr.astype(jnp.bfloat16)
    for g in range(IH):
        qi4_ref[g] = _f32dot(xb, wqi_ref[g]).astype(jnp.bfloat16)
    ki_ref[...] = _f32dot(xb, wki_ref[...]).astype(jnp.bfloat16)


def _proj_kv(hs_ref, wkva_ref, ksc_ref, wkvb_ref, cos_ref, sin_ref,
             k4_ref, v4_ref):
    xb = hs_ref[...].astype(jnp.bfloat16)
    kva = _f32dot(xb, wkva_ref[...])
    ckv = kva[:, :KVLORA]
    kr = kva[:, KVLORA:]
    var = jnp.mean(ckv * ckv, axis=1, keepdims=True)
    cn = ckv * jax.lax.rsqrt(var + EPS) * ksc_ref[...]
    cnb = cn.astype(jnp.bfloat16)
    c, s = cos_ref[...], sin_ref[...]
    kr = kr * c + _pairswap(kr) * s
    krb = kr.astype(jnp.bfloat16)
    for h in range(H):
        kvh = _f32dot(cnb, wkvb_ref[h])
        k4_ref[h, :, :NOPE] = kvh[:, :NOPE].astype(jnp.bfloat16)
        k4_ref[h, :, NOPE:] = krb
        v4_ref[h] = kvh[:, NOPE:].astype(jnp.bfloat16)


def _make_select(width, c0, tie_iters):
    def _select(qi_ref, ki_ref, w_ref, m_ref):
        ki = ki_ref[...]
        sc = jnp.zeros((QB, width), jnp.float32)
        for g in range(IH):
            r = jnp.maximum(_f32dot_nt(qi_ref[g], ki), 0.0)
            sc = sc + w_ref[0, g] * r.astype(jnp.bfloat16).astype(jnp.float32)
        pid = pl.program_id(0)
        col = jax.lax.broadcasted_iota(jnp.int32, (QB, width), 1)
        row = (c0 + pid) * QB + jax.lax.broadcasted_iota(
            jnp.int32, (QB, width), 0)
        causal = col <= row
        key = jnp.where(causal, jax.lax.bitcast_convert_type(sc, jnp.int32),
                        jnp.int32(-1))
        lo = jnp.full((QB, 1), -1, jnp.int32)
        hi = jnp.max(key, axis=1, keepdims=True) + 1
        for _ in range(32):
            mid = lo + jax.lax.shift_right_logical(hi - lo, 1)
            cnt = jnp.sum((key >= mid).astype(jnp.int32), axis=1,
                          keepdims=True)
            ok = cnt >= TOPK
            lo = jnp.where(ok, mid, lo)
            hi = jnp.where(ok, hi, mid)
        t = lo
        cgt = jnp.sum((key > t).astype(jnp.int32), axis=1, keepdims=True)
        need = TOPK - cgt
        eq = key == t
        tlo = jnp.zeros((QB, 1), jnp.int32)
        thi = jnp.full((QB, 1), width, jnp.int32)
        for _ in range(tie_iters):
            mid = tlo + jax.lax.shift_right_logical(thi - tlo, 1)
            cnt = jnp.sum((eq & (col < mid)).astype(jnp.int32),
                          axis=1, keepdims=True)
            ok = cnt >= need
            thi = jnp.where(ok, mid, thi)
            tlo = jnp.where(ok, tlo, mid)
        sel = ((key > t) | (eq & (col < thi))) & causal
        m_ref[:, :width] = sel.astype(jnp.int8)
        if width < S:
            m_ref[:, width:] = jnp.zeros((QB, S - width), jnp.int8)
    return _select


def _attn(q_ref, k_ref, v_ref, m_ref, o_ref):
    m = m_ref[...] != 0
    for hh in range(2):
        sc = _f32dot_nt(q_ref[hh], k_ref[hh])
        sc = jnp.where(m, sc * SCALE, NEG)
        mx = jnp.max(sc, axis=1, keepdims=True)
        p = jnp.exp(sc - mx)
        den = jnp.sum(p, axis=1, keepdims=True)
        pb = (p / den).astype(jnp.bfloat16)
        o_ref[:, hh * VDIM:(hh + 1) * VDIM] = (
            _f32dot(pb, v_ref[hh]).astype(jnp.bfloat16))


def _outproj(a_ref, wo_ref, o_ref):
    o_ref[...] = _f32dot(a_ref[...], wo_ref[...])


def _full(shape):
    n = len(shape)
    return pl.BlockSpec(shape, lambda *_: (0,) * n)


@jax.jit
def kernel(hidden_states, position_ids, W_qa, q_a_scale, W_qb, W_kva,
           kv_a_scale, W_kvb, W_o, W_q_idx, W_k_idx, w_idx):
    hs = hidden_states.reshape(S, HID)
    bf16 = jnp.bfloat16
    wqa_b = W_qa.astype(bf16)
    wkva_b = W_kva.astype(bf16)
    wki_b = W_k_idx.astype(bf16)
    wo_b = W_o.astype(bf16)
    wqb4 = W_qb.astype(bf16).reshape(QLORA, H, NOPE + ROPE).transpose(1, 0, 2)
    wkvb4 = W_kvb.astype(bf16).reshape(KVLORA, H, NOPE + VDIM).transpose(1, 0, 2)
    wqi4 = W_q_idx.astype(bf16).reshape(HID, IH, IDIM).transpose(1, 0, 2)
    qsc = q_a_scale.reshape(1, QLORA)
    ksc = kv_a_scale.reshape(1, KVLORA)
    w2 = w_idx.reshape(1, IH)
    inv = 1.0 / (10000.0 ** (np.arange(0, ROPE, 2, dtype=np.float32) / ROPE))
    inv2 = jnp.asarray(np.repeat(inv, 2).reshape(1, ROPE))
    fr = position_ids.reshape(S, 1).astype(jnp.float32) * inv2
    cos_t = jnp.cos(fr)
    sin_t = jnp.sin(fr)
    nq = S // QB
    f32 = jnp.float32

    q4, qi4, ki = pl.pallas_call(
        _proj_q,
        grid=(nq,),
        in_specs=[
            pl.BlockSpec((QB, HID), lambda i: (i, 0)),
            _full((HID, QLORA)),
            _full((1, QLORA)),
            _full((H, QLORA, NOPE + ROPE)),
            _full((IH, HID, IDIM)),
            _full((HID, IDIM)),
            pl.BlockSpec((QB, ROPE), lambda i: (i, 0)),
            pl.BlockSpec((QB, ROPE), lambda i: (i, 0)),
        ],
        out_specs=[
            pl.BlockSpec((H, QB, NOPE + ROPE), lambda i: (0, i, 0)),
            pl.BlockSpec((IH, QB, IDIM), lambda i: (0, i, 0)),
            pl.BlockSpec((QB, IDIM), lambda i: (i, 0)),
        ],
        out_shape=[
            jax.ShapeDtypeStruct((H, S, NOPE + ROPE), bf16),
            jax.ShapeDtypeStruct((IH, S, IDIM), bf16),
            jax.ShapeDtypeStruct((S, IDIM), bf16),
        ],
    )(hs, wqa_b, qsc, wqb4, wqi4, wki_b, cos_t, sin_t)

    k4, v4 = pl.pallas_call(
        _proj_kv,
        grid=(nq,),
        in_specs=[
            pl.BlockSpec((QB, HID), lambda i: (i, 0)),
            _full((HID, KVLORA + ROPE)),
            _full((1, KVLORA)),
            _full((H, KVLORA, NOPE + VDIM)),
            pl.BlockSpec((QB, ROPE), lambda i: (i, 0)),
            pl.BlockSpec((QB, ROPE), lambda i: (i, 0)),
        ],
        out_specs=[
            pl.BlockSpec((H, QB, NOPE + ROPE), lambda i: (0, i, 0)),
            pl.BlockSpec((H, QB, VDIM), lambda i: (0, i, 0)),
        ],
        out_shape=[
            jax.ShapeDtypeStruct((H, S, NOPE + ROPE), bf16),
            jax.ShapeDtypeStruct((H, S, VDIM), bf16),
        ],
    )(hs, wkva_b, ksc, wkvb4, cos_t, sin_t)

    masks = []
    for ci in range(4):
        width = (ci + 1) * 2 * QB
        tie_iters = max(1, int(np.ceil(np.log2(width))))
        masks.append(pl.pallas_call(
            _make_select(width, ci * 2, tie_iters),
            grid=(2,),
            in_specs=[
                pl.BlockSpec((IH, QB, IDIM),
                             lambda i, c=ci: (0, c * 2 + i, 0)),
                pl.BlockSpec((width, IDIM), lambda i: (0, 0)),
                _full((1, IH)),
            ],
            out_specs=pl.BlockSpec((QB, S), lambda i: (i, 0)),
            out_shape=jax.ShapeDtypeStruct((2 * QB, S), jnp.int8),
        )(qi4, ki, w2))
    mask = jnp.concatenate(masks, axis=0)

    att2 = pl.pallas_call(
        _attn,
        grid=(H // 2, nq),
        in_specs=[
            pl.BlockSpec((2, QB, NOPE + ROPE), lambda h, i: (h, i, 0)),
            pl.BlockSpec((2, S, NOPE + ROPE), lambda h, i: (h, 0, 0)),
            pl.BlockSpec((2, S, VDIM), lambda h, i: (h, 0, 0)),
            pl.BlockSpec((QB, S), lambda h, i: (i, 0)),
        ],
        out_specs=pl.BlockSpec((QB, 2 * VDIM), lambda h, i: (i, h)),
        out_shape=jax.ShapeDtypeStruct((S, H * VDIM), bf16),
    )(q4, k4, v4, mask)

    out = pl.pallas_call(
        _outproj,
        grid=(nq,),
        in_specs=[
            pl.BlockSpec((QB, H * VDIM), lambda i: (i, 0)),
            _full((H * VDIM, HID)),
        ],
        out_specs=pl.BlockSpec((QB, HID), lambda i: (i, 0)),
        out_shape=jax.ShapeDtypeStruct((S, HID), f32),
    )(att2, wo_b)

    return out.reshape(1, S, HID)


# speedup vs baseline: 14.4743x; 1.0700x over previous
import numpy as np
import jax
import jax.numpy as jnp
from jax.experimental import pallas as pl
from jax.experimental.pallas import tpu as pltpu

S, HID = 2048, 2048
H, NOPE, ROPE, VDIM = 16, 128, 64, 128
QLORA, KVLORA = 1024, 512
IH, IDIM, TOPK = 4, 64, 512
EPS = 1e-5
SCALE = 1.0 / float(np.sqrt(NOPE + ROPE))
QB = 256
NEG = -1e30


def _pairswap(x):
    even = jax.lax.broadcasted_iota(jnp.int32, x.shape, 1) % 2 == 0
    n = x.shape[1]
    return jnp.where(even, -pltpu.roll(x, n - 1, 1), pltpu.roll(x, 1, 1))


def _f32dot(a, b):
    return jnp.dot(a, b, preferred_element_type=jnp.float32)


def _f32dot_nt(a, b):
    return jax.lax.dot_general(
        a, b, (((1,), (1,)), ((), ())), preferred_element_type=jnp.float32)


def _proj(hs_ref, wqa_ref, qsc_ref, wqb_ref, wqi_ref, wki_ref,
          wkva_ref, ksc_ref, wkvb_ref, cos_ref, sin_ref,
          q4_ref, qi4_ref, ki_ref, k4_ref, v4_ref):
    xb = hs_ref[...].astype(jnp.bfloat16)
    c, s = cos_ref[...], sin_ref[...]
    qa = _f32dot(xb, wqa_ref[...])
    var = jnp.mean(qa * qa, axis=1, keepdims=True)
    qn = qa * jax.lax.rsqrt(var + EPS) * qsc_ref[...]
    qnb = qn.astype(jnp.bfloat16)
    kva = _f32dot(xb, wkva_ref[...])
    ckv = kva[:, :KVLORA]
    kr = kva[:, KVLORA:]
    varc = jnp.mean(ckv * ckv, axis=1, keepdims=True)
    cn = ckv * jax.lax.rsqrt(varc + EPS) * ksc_ref[...]
    cnb = cn.astype(jnp.bfloat16)
    kr = kr * c + _pairswap(kr) * s
    krb = kr.astype(jnp.bfloat16)
    for h in range(H):
        qh = _f32dot(qnb, wqb_ref[h])
        qr = qh[:, NOPE:]
        qr = qr * c + _pairswap(qr) * s
        q4_ref[h, :, :NOPE] = qh[:, :NOPE].astype(jnp.bfloat16)
        q4_ref[h, :, NOPE:] = qr.astype(jnp.bfloat16)
        kvh = _f32dot(cnb, wkvb_ref[h])
        k4_ref[h, :, :NOPE] = kvh[:, :NOPE].astype(jnp.bfloat16)
        k4_ref[h, :, NOPE:] = krb
        v4_ref[h] = kvh[:, NOPE:].astype(jnp.bfloat16)
    for g in range(IH):
        qi4_ref[g] = _f32dot(xb, wqi_ref[g]).astype(jnp.bfloat16)
    ki_ref[...] = _f32dot(xb, wki_ref[...]).astype(jnp.bfloat16)


def _make_select(width, c0, tie_iters):
    def _select(qi_ref, ki_ref, w_ref, m_ref):
        ki = ki_ref[...]
        sc = jnp.zeros((QB, width), jnp.float32)
        for g in range(IH):
            r = jnp.maximum(_f32dot_nt(qi_ref[g], ki), 0.0)
            sc = sc + w_ref[0, g] * r.astype(jnp.bfloat16).astype(jnp.float32)
        pid = pl.program_id(0)
        col = jax.lax.broadcasted_iota(jnp.int32, (QB, width), 1)
        row = (c0 + pid) * QB + jax.lax.broadcasted_iota(
            jnp.int32, (QB, width), 0)
        causal = col <= row
        key = jnp.where(causal, jax.lax.bitcast_convert_type(sc, jnp.int32),
                        jnp.int32(-1))
        lo = jnp.full((QB, 1), -1, jnp.int32)
        hi = jnp.max(key, axis=1, keepdims=True) + 1
        for _ in range(32):
            mid = lo + jax.lax.shift_right_logical(hi - lo, 1)
            cnt = jnp.sum((key >= mid).astype(jnp.int32), axis=1,
                          keepdims=True)
            ok = cnt >= TOPK
            lo = jnp.where(ok, mid, lo)
            hi = jnp.where(ok, hi, mid)
        t = lo
        cgt = jnp.sum((key > t).astype(jnp.int32), axis=1, keepdims=True)
        need = TOPK - cgt
        eq = key == t
        tlo = jnp.zeros((QB, 1), jnp.int32)
        thi = jnp.full((QB, 1), width, jnp.int32)
        for _ in range(tie_iters):
            mid = tlo + jax.lax.shift_right_logical(thi - tlo, 1)
            cnt = jnp.sum((eq & (col < mid)).astype(jnp.int32),
                          axis=1, keepdims=True)
            ok = cnt >= need
            thi = jnp.where(ok, mid, thi)
            tlo = jnp.where(ok, tlo, mid)
        sel = ((key > t) | (eq & (col < thi))) & causal
        m_ref[:, :width] = sel.astype(jnp.int8)
        if width < S:
            m_ref[:, width:] = jnp.zeros((QB, S - width), jnp.int8)
    return _select


def _attn(q_ref, k_ref, v_ref, m_ref, o_ref):
    m = m_ref[...] != 0
    for hh in range(4):
        sc = _f32dot_nt(q_ref[hh], k_ref[hh])
        sc = jnp.where(m, sc * SCALE, NEG)
        mx = jnp.max(sc, axis=1, keepdims=True)
        p = jnp.exp(sc - mx)
        den = jnp.sum(p, axis=1, keepdims=True)
        pb = (p / den).astype(jnp.bfloat16)
        o_ref[:, hh * VDIM:(hh + 1) * VDIM] = (
            _f32dot(pb, v_ref[hh]).astype(jnp.bfloat16))


def _outproj(a_ref, wo_ref, o_ref):
    o_ref[...] = _f32dot(a_ref[...], wo_ref[...])


def _full(shape):
    n = len(shape)
    return pl.BlockSpec(shape, lambda *_: (0,) * n)


@jax.jit
def kernel(hidden_states, position_ids, W_qa, q_a_scale, W_qb, W_kva,
           kv_a_scale, W_kvb, W_o, W_q_idx, W_k_idx, w_idx):
    hs = hidden_states.reshape(S, HID)
    bf16 = jnp.bfloat16
    wqa_b = W_qa.astype(bf16)
    wkva_b = W_kva.astype(bf16)
    wki_b = W_k_idx.astype(bf16)
    wo_b = W_o.astype(bf16)
    wqb4 = W_qb.astype(bf16).reshape(QLORA, H, NOPE + ROPE).transpose(1, 0, 2)
    wkvb4 = W_kvb.astype(bf16).reshape(KVLORA, H, NOPE + VDIM).transpose(1, 0, 2)
    wqi4 = W_q_idx.astype(bf16).reshape(HID, IH, IDIM).transpose(1, 0, 2)
    qsc = q_a_scale.reshape(1, QLORA)
    ksc = kv_a_scale.reshape(1, KVLORA)
    w2 = w_idx.reshape(1, IH)
    inv = 1.0 / (10000.0 ** (np.arange(0, ROPE, 2, dtype=np.float32) / ROPE))
    inv2 = jnp.asarray(np.repeat(inv, 2).reshape(1, ROPE))
    fr = position_ids.reshape(S, 1).astype(jnp.float32) * inv2
    cos_t = jnp.cos(fr)
    sin_t = jnp.sin(fr)
    nq = S // QB
    f32 = jnp.float32

    PB = 512
    q4, qi4, ki, k4, v4 = pl.pallas_call(
        _proj,
        grid=(S // PB,),
        in_specs=[
            pl.BlockSpec((PB, HID), lambda i: (i, 0)),
            _full((HID, QLORA)),
            _full((1, QLORA)),
            _full((H, QLORA, NOPE + ROPE)),
            _full((IH, HID, IDIM)),
            _full((HID, IDIM)),
            _full((HID, KVLORA + ROPE)),
            _full((1, KVLORA)),
            _full((H, KVLORA, NOPE + VDIM)),
            pl.BlockSpec((PB, ROPE), lambda i: (i, 0)),
            pl.BlockSpec((PB, ROPE), lambda i: (i, 0)),
        ],
        out_specs=[
            pl.BlockSpec((H, PB, NOPE + ROPE), lambda i: (0, i, 0)),
            pl.BlockSpec((IH, PB, IDIM), lambda i: (0, i, 0)),
            pl.BlockSpec((PB, IDIM), lambda i: (i, 0)),
            pl.BlockSpec((H, PB, NOPE + ROPE), lambda i: (0, i, 0)),
            pl.BlockSpec((H, PB, VDIM), lambda i: (0, i, 0)),
        ],
        out_shape=[
            jax.ShapeDtypeStruct((H, S, NOPE + ROPE), bf16),
            jax.ShapeDtypeStruct((IH, S, IDIM), bf16),
            jax.ShapeDtypeStruct((S, IDIM), bf16),
            jax.ShapeDtypeStruct((H, S, NOPE + ROPE), bf16),
            jax.ShapeDtypeStruct((H, S, VDIM), bf16),
        ],
    )(hs, wqa_b, qsc, wqb4, wqi4, wki_b, wkva_b, ksc, wkvb4, cos_t, sin_t)

    masks = []
    for ci in range(4):
        width = (ci + 1) * 2 * QB
        tie_iters = max(1, int(np.ceil(np.log2(width))))
        masks.append(pl.pallas_call(
            _make_select(width, ci * 2, tie_iters),
            grid=(2,),
            in_specs=[
                pl.BlockSpec((IH, QB, IDIM),
                             lambda i, c=ci: (0, c * 2 + i, 0)),
                pl.BlockSpec((width, IDIM), lambda i: (0, 0)),
                _full((1, IH)),
            ],
            out_specs=pl.BlockSpec((QB, S), lambda i: (i, 0)),
            out_shape=jax.ShapeDtypeStruct((2 * QB, S), jnp.int8),
        )(qi4, ki, w2))
    mask = jnp.concatenate(masks, axis=0)

    att2 = pl.pallas_call(
        _attn,
        grid=(H // 4, nq),
        in_specs=[
            pl.BlockSpec((4, QB, NOPE + ROPE), lambda h, i: (h, i, 0)),
            pl.BlockSpec((4, S, NOPE + ROPE), lambda h, i: (h, 0, 0)),
            pl.BlockSpec((4, S, VDIM), lambda h, i: (h, 0, 0)),
            pl.BlockSpec((QB, S), lambda h, i: (i, 0)),
        ],
        out_specs=pl.BlockSpec((QB, 4 * VDIM), lambda h, i: (i, h)),
        out_shape=jax.ShapeDtypeStruct((S, H * VDIM), bf16),
    )(q4, k4, v4, mask)

    out = pl.pallas_call(
        _outproj,
        grid=(nq,),
        in_specs=[
            pl.BlockSpec((QB, H * VDIM), lambda i: (i, 0)),
            _full((H * VDIM, HID)),
        ],
        out_specs=pl.BlockSpec((QB, HID), lambda i: (i, 0)),
        out_shape=jax.ShapeDtypeStruct((S, HID), f32),
    )(att2, wo_b)

    return out.reshape(1, S, HID)


# speedup vs baseline: 15.4223x; 1.0655x over previous
import numpy as np
import jax
import jax.numpy as jnp
from jax.experimental import pallas as pl
from jax.experimental.pallas import tpu as pltpu

S, HID = 2048, 2048
H, NOPE, ROPE, VDIM = 16, 128, 64, 128
QLORA, KVLORA = 1024, 512
IH, IDIM, TOPK = 4, 64, 512
EPS = 1e-5
SCALE = 1.0 / float(np.sqrt(NOPE + ROPE))
QB = 256
NEG = -1e30


def _pairswap(x):
    even = jax.lax.broadcasted_iota(jnp.int32, x.shape, 1) % 2 == 0
    n = x.shape[1]
    return jnp.where(even, -pltpu.roll(x, n - 1, 1), pltpu.roll(x, 1, 1))


def _f32dot(a, b):
    return jnp.dot(a, b, preferred_element_type=jnp.float32)


def _f32dot_nt(a, b):
    return jax.lax.dot_general(
        a, b, (((1,), (1,)), ((), ())), preferred_element_type=jnp.float32)


def _proj(hs_ref, wqa_ref, qsc_ref, wqb_ref, wqi_ref, wki_ref,
          wkva_ref, ksc_ref, wkvb_ref, cos_ref, sin_ref,
          q4_ref, qi4_ref, ki_ref, k4_ref, v4_ref):
    xb = hs_ref[...].astype(jnp.bfloat16)
    c, s = cos_ref[...], sin_ref[...]
    qa = _f32dot(xb, wqa_ref[...])
    var = jnp.mean(qa * qa, axis=1, keepdims=True)
    qn = qa * jax.lax.rsqrt(var + EPS) * qsc_ref[...]
    qnb = qn.astype(jnp.bfloat16)
    kva = _f32dot(xb, wkva_ref[...])
    ckv = kva[:, :KVLORA]
    kr = kva[:, KVLORA:]
    varc = jnp.mean(ckv * ckv, axis=1, keepdims=True)
    cn = ckv * jax.lax.rsqrt(varc + EPS) * ksc_ref[...]
    cnb = cn.astype(jnp.bfloat16)
    kr = kr * c + _pairswap(kr) * s
    krb = kr.astype(jnp.bfloat16)
    for h in range(H):
        qh = _f32dot(qnb, wqb_ref[h])
        qr = qh[:, NOPE:]
        qr = qr * c + _pairswap(qr) * s
        q4_ref[h, :, :NOPE] = qh[:, :NOPE].astype(jnp.bfloat16)
        q4_ref[h, :, NOPE:] = qr.astype(jnp.bfloat16)
        kvh = _f32dot(cnb, wkvb_ref[h])
        k4_ref[h, :, :NOPE] = kvh[:, :NOPE].astype(jnp.bfloat16)
        k4_ref[h, :, NOPE:] = krb
        v4_ref[h] = kvh[:, NOPE:].astype(jnp.bfloat16)
    for g in range(IH):
        qi4_ref[g] = _f32dot(xb, wqi_ref[g]).astype(jnp.bfloat16)
    ki_ref[...] = _f32dot(xb, wki_ref[...]).astype(jnp.bfloat16)


def _make_select(width, c0, tie_iters):
    def _select(qi_ref, ki_ref, w_ref, m_ref):
        ki = ki_ref[...]
        sc = jnp.zeros((QB, width), jnp.float32)
        for g in range(IH):
            r = jnp.maximum(_f32dot_nt(qi_ref[g], ki), 0.0)
            sc = sc + w_ref[0, g] * r.astype(jnp.bfloat16).astype(jnp.float32)
        pid = pl.program_id(0)
        col = jax.lax.broadcasted_iota(jnp.int32, (QB, width), 1)
        row = (c0 + pid) * QB + jax.lax.broadcasted_iota(
            jnp.int32, (QB, width), 0)
        causal = col <= row
        key = jnp.where(causal, jax.lax.bitcast_convert_type(sc, jnp.int32),
                        jnp.int32(-1))
        lo = jnp.full((QB, 1), -1, jnp.int32)
        hi = jnp.max(key, axis=1, keepdims=True) + 1
        for _ in range(32):
            mid = lo + jax.lax.shift_right_logical(hi - lo, 1)
            cnt = jnp.sum((key >= mid).astype(jnp.int32), axis=1,
                          keepdims=True)
            ok = cnt >= TOPK
            lo = jnp.where(ok, mid, lo)
            hi = jnp.where(ok, hi, mid)
        t = lo
        cgt = jnp.sum((key > t).astype(jnp.int32), axis=1, keepdims=True)
        need = TOPK - cgt
        eq = key == t
        tlo = jnp.zeros((QB, 1), jnp.int32)
        thi = jnp.full((QB, 1), width, jnp.int32)
        for _ in range(tie_iters):
            mid = tlo + jax.lax.shift_right_logical(thi - tlo, 1)
            cnt = jnp.sum((eq & (col < mid)).astype(jnp.int32),
                          axis=1, keepdims=True)
            ok = cnt >= need
            thi = jnp.where(ok, mid, thi)
            tlo = jnp.where(ok, tlo, mid)
        sel = ((key > t) | (eq & (col < thi))) & causal
        m_ref[:, :width] = sel.astype(jnp.int8)
        if width < S:
            m_ref[:, width:] = jnp.zeros((QB, S - width), jnp.int8)
    return _select


def _make_attn(width):
    def _attn(q_ref, k_ref, v_ref, m_ref, o_ref):
        m = m_ref[...] != 0
        for hh in range(4):
            sc = _f32dot_nt(q_ref[hh], k_ref[hh])
            sc = jnp.where(m, sc * SCALE, NEG)
            mx = jnp.max(sc, axis=1, keepdims=True)
            p = jnp.exp(sc - mx)
            den = jnp.sum(p, axis=1, keepdims=True)
            pb = (p / den).astype(jnp.bfloat16)
            o_ref[:, hh * VDIM:(hh + 1) * VDIM] = (
                _f32dot(pb, v_ref[hh]).astype(jnp.bfloat16))
    return _attn


def _outproj(a_ref, wo_ref, o_ref):
    o_ref[...] = _f32dot(a_ref[...], wo_ref[...])


def _full(shape):
    n = len(shape)
    return pl.BlockSpec(shape, lambda *_: (0,) * n)


@jax.jit
def kernel(hidden_states, position_ids, W_qa, q_a_scale, W_qb, W_kva,
           kv_a_scale, W_kvb, W_o, W_q_idx, W_k_idx, w_idx):
    hs = hidden_states.reshape(S, HID)
    bf16 = jnp.bfloat16
    wqa_b = W_qa.astype(bf16)
    wkva_b = W_kva.astype(bf16)
    wki_b = W_k_idx.astype(bf16)
    wo_b = W_o.astype(bf16)
    wqb4 = W_qb.astype(bf16).reshape(QLORA, H, NOPE + ROPE).transpose(1, 0, 2)
    wkvb4 = W_kvb.astype(bf16).reshape(KVLORA, H, NOPE + VDIM).transpose(1, 0, 2)
    wqi4 = W_q_idx.astype(bf16).reshape(HID, IH, IDIM).transpose(1, 0, 2)
    qsc = q_a_scale.reshape(1, QLORA)
    ksc = kv_a_scale.reshape(1, KVLORA)
    w2 = w_idx.reshape(1, IH)
    inv = 1.0 / (10000.0 ** (np.arange(0, ROPE, 2, dtype=np.float32) / ROPE))
    inv2 = jnp.asarray(np.repeat(inv, 2).reshape(1, ROPE))
    fr = position_ids.reshape(S, 1).astype(jnp.float32) * inv2
    cos_t = jnp.cos(fr)
    sin_t = jnp.sin(fr)
    nq = S // QB
    f32 = jnp.float32

    PB = 512
    q4, qi4, ki, k4, v4 = pl.pallas_call(
        _proj,
        grid=(S // PB,),
        in_specs=[
            pl.BlockSpec((PB, HID), lambda i: (i, 0)),
            _full((HID, QLORA)),
            _full((1, QLORA)),
            _full((H, QLORA, NOPE + ROPE)),
            _full((IH, HID, IDIM)),
            _full((HID, IDIM)),
            _full((HID, KVLORA + ROPE)),
            _full((1, KVLORA)),
            _full((H, KVLORA, NOPE + VDIM)),
            pl.BlockSpec((PB, ROPE), lambda i: (i, 0)),
            pl.BlockSpec((PB, ROPE), lambda i: (i, 0)),
        ],
        out_specs=[
            pl.BlockSpec((H, PB, NOPE + ROPE), lambda i: (0, i, 0)),
            pl.BlockSpec((IH, PB, IDIM), lambda i: (0, i, 0)),
            pl.BlockSpec((PB, IDIM), lambda i: (i, 0)),
            pl.BlockSpec((H, PB, NOPE + ROPE), lambda i: (0, i, 0)),
            pl.BlockSpec((H, PB, VDIM), lambda i: (0, i, 0)),
        ],
        out_shape=[
            jax.ShapeDtypeStruct((H, S, NOPE + ROPE), bf16),
            jax.ShapeDtypeStruct((IH, S, IDIM), bf16),
            jax.ShapeDtypeStruct((S, IDIM), bf16),
            jax.ShapeDtypeStruct((H, S, NOPE + ROPE), bf16),
            jax.ShapeDtypeStruct((H, S, VDIM), bf16),
        ],
    )(hs, wqa_b, qsc, wqb4, wqi4, wki_b, wkva_b, ksc, wkvb4, cos_t, sin_t)

    masks = []
    for ci in range(4):
        width = (ci + 1) * 2 * QB
        tie_iters = max(1, int(np.ceil(np.log2(width))))
        masks.append(pl.pallas_call(
            _make_select(width, ci * 2, tie_iters),
            grid=(2,),
            in_specs=[
                pl.BlockSpec((IH, QB, IDIM),
                             lambda i, c=ci: (0, c * 2 + i, 0)),
                pl.BlockSpec((width, IDIM), lambda i: (0, 0)),
                _full((1, IH)),
            ],
            out_specs=pl.BlockSpec((QB, S), lambda i: (i, 0)),
            out_shape=jax.ShapeDtypeStruct((2 * QB, S), jnp.int8),
        )(qi4, ki, w2))
    mask = jnp.concatenate(masks, axis=0)

    att_parts = []
    for ci in range(4):
        width = (ci + 1) * 2 * QB
        att_parts.append(pl.pallas_call(
            _make_attn(width),
            grid=(H // 4, 2),
            in_specs=[
                pl.BlockSpec((4, QB, NOPE + ROPE),
                             lambda h, i, c=ci: (h, c * 2 + i, 0)),
                pl.BlockSpec((4, width, NOPE + ROPE), lambda h, i: (h, 0, 0)),
                pl.BlockSpec((4, width, VDIM), lambda h, i: (h, 0, 0)),
                pl.BlockSpec((QB, width), lambda h, i, c=ci: (c * 2 + i, 0)),
            ],
            out_specs=pl.BlockSpec((QB, 4 * VDIM), lambda h, i: (i, h)),
            out_shape=jax.ShapeDtypeStruct((2 * QB, H * VDIM), bf16),
        )(q4, k4, v4, mask))
    att2 = jnp.concatenate(att_parts, axis=0)

    out = pl.pallas_call(
        _outproj,
        grid=(nq,),
        in_specs=[
            pl.BlockSpec((QB, H * VDIM), lambda i: (i, 0)),
            _full((H * VDIM, HID)),
        ],
        out_specs=pl.BlockSpec((QB, HID), lambda i: (i, 0)),
        out_shape=jax.ShapeDtypeStruct((S, HID), f32),
    )(att2, wo_b)

    return out.reshape(1, S, HID)


# speedup vs baseline: 16.5121x; 1.0707x over previous
import numpy as np
import jax
import jax.numpy as jnp
from jax.experimental import pallas as pl
from jax.experimental.pallas import tpu as pltpu

S, HID = 2048, 2048
H, NOPE, ROPE, VDIM = 16, 128, 64, 128
QLORA, KVLORA = 1024, 512
IH, IDIM, TOPK = 4, 64, 512
EPS = 1e-5
SCALE = 1.0 / float(np.sqrt(NOPE + ROPE))
QB = 256
NEG = -1e30


def _pairswap(x):
    even = jax.lax.broadcasted_iota(jnp.int32, x.shape, 1) % 2 == 0
    n = x.shape[1]
    return jnp.where(even, -pltpu.roll(x, n - 1, 1), pltpu.roll(x, 1, 1))


def _f32dot(a, b):
    return jnp.dot(a, b, preferred_element_type=jnp.float32)


def _f32dot_nt(a, b):
    return jax.lax.dot_general(
        a, b, (((1,), (1,)), ((), ())), preferred_element_type=jnp.float32)


def _proj(hs_ref, wqa_ref, qsc_ref, wqb_ref, wqi_ref, wki_ref,
          wkva_ref, ksc_ref, wkvb_ref, cos_ref, sin_ref,
          q4_ref, qi4_ref, ki_ref, k4_ref, v4_ref):
    xb = hs_ref[...].astype(jnp.bfloat16)
    c, s = cos_ref[...], sin_ref[...]
    qa = _f32dot(xb, wqa_ref[...])
    var = jnp.mean(qa * qa, axis=1, keepdims=True)
    qn = qa * jax.lax.rsqrt(var + EPS) * qsc_ref[...]
    qnb = qn.astype(jnp.bfloat16)
    kva = _f32dot(xb, wkva_ref[...])
    ckv = kva[:, :KVLORA]
    kr = kva[:, KVLORA:]
    varc = jnp.mean(ckv * ckv, axis=1, keepdims=True)
    cn = ckv * jax.lax.rsqrt(varc + EPS) * ksc_ref[...]
    cnb = cn.astype(jnp.bfloat16)
    kr = kr * c + _pairswap(kr) * s
    krb = kr.astype(jnp.bfloat16)
    for h in range(H):
        qh = _f32dot(qnb, wqb_ref[h])
        qr = qh[:, NOPE:]
        qr = qr * c + _pairswap(qr) * s
        q4_ref[h, :, :NOPE] = qh[:, :NOPE].astype(jnp.bfloat16)
        q4_ref[h, :, NOPE:] = qr.astype(jnp.bfloat16)
        kvh = _f32dot(cnb, wkvb_ref[h])
        k4_ref[h, :, :NOPE] = kvh[:, :NOPE].astype(jnp.bfloat16)
        k4_ref[h, :, NOPE:] = krb
        v4_ref[h] = kvh[:, NOPE:].astype(jnp.bfloat16)
    for g in range(IH):
        qi4_ref[g] = _f32dot(xb, wqi_ref[g]).astype(jnp.bfloat16)
    ki_ref[...] = _f32dot(xb, wki_ref[...]).astype(jnp.bfloat16)


SB = 512


def _make_select(width, c0, tie_iters):
    def _select(qi_ref, ki_ref, w_ref, m_ref):
        ki = ki_ref[...]
        sc = jnp.zeros((SB, width), jnp.float32)
        for g in range(IH):
            r = jnp.maximum(_f32dot_nt(qi_ref[g], ki), 0.0)
            sc = sc + w_ref[0, g] * r.astype(jnp.bfloat16).astype(jnp.float32)
        col = jax.lax.broadcasted_iota(jnp.int32, (SB, width), 1)
        row = c0 * QB + jax.lax.broadcasted_iota(
            jnp.int32, (SB, width), 0)
        causal = col <= row
        key = jnp.where(causal, jax.lax.bitcast_convert_type(sc, jnp.int32),
                        jnp.int32(-1))
        lo = jnp.full((SB, 1), -1, jnp.int32)
        hi = jnp.max(key, axis=1, keepdims=True) + 1
        for _ in range(32):
            mid = lo + jax.lax.shift_right_logical(hi - lo, 1)
            cnt = jnp.sum((key >= mid).astype(jnp.int32), axis=1,
                          keepdims=True)
            ok = cnt >= TOPK
            lo = jnp.where(ok, mid, lo)
            hi = jnp.where(ok, hi, mid)
        t = lo
        cgt = jnp.sum((key > t).astype(jnp.int32), axis=1, keepdims=True)
        need = TOPK - cgt
        eq = key == t
        tlo = jnp.zeros((SB, 1), jnp.int32)
        thi = jnp.full((SB, 1), width, jnp.int32)
        for _ in range(tie_iters):
            mid = tlo + jax.lax.shift_right_logical(thi - tlo, 1)
            cnt = jnp.sum((eq & (col < mid)).astype(jnp.int32),
                          axis=1, keepdims=True)
            ok = cnt >= need
            thi = jnp.where(ok, mid, thi)
            tlo = jnp.where(ok, tlo, mid)
        sel = ((key > t) | (eq & (col < thi))) & causal
        m_ref[:, :width] = sel.astype(jnp.int8)
        if width < S:
            m_ref[:, width:] = jnp.zeros((SB, S - width), jnp.int8)
    return _select


def _make_attn(width):
    def _attn(q_ref, k_ref, v_ref, m_ref, o_ref):
        m = m_ref[...] != 0
        for hh in range(4):
            sc = _f32dot_nt(q_ref[hh], k_ref[hh])
            sc = jnp.where(m, sc * SCALE, NEG)
            mx = jnp.max(sc, axis=1, keepdims=True)
            p = jnp.exp(sc - mx)
            den = jnp.sum(p, axis=1, keepdims=True)
            pb = (p / den).astype(jnp.bfloat16)
            o_ref[:, hh * VDIM:(hh + 1) * VDIM] = (
                _f32dot(pb, v_ref[hh]).astype(jnp.bfloat16))
    return _attn


def _outproj(a_ref, wo_ref, o_ref):
    o_ref[...] = _f32dot(a_ref[...], wo_ref[...])


def _full(shape):
    n = len(shape)
    return pl.BlockSpec(shape, lambda *_: (0,) * n)


@jax.jit
def kernel(hidden_states, position_ids, W_qa, q_a_scale, W_qb, W_kva,
           kv_a_scale, W_kvb, W_o, W_q_idx, W_k_idx, w_idx):
    hs = hidden_states.reshape(S, HID)
    bf16 = jnp.bfloat16
    wqa_b = W_qa.astype(bf16)
    wkva_b = W_kva.astype(bf16)
    wki_b = W_k_idx.astype(bf16)
    wo_b = W_o.astype(bf16)
    wqb4 = W_qb.astype(bf16).reshape(QLORA, H, NOPE + ROPE).transpose(1, 0, 2)
    wkvb4 = W_kvb.astype(bf16).reshape(KVLORA, H, NOPE + VDIM).transpose(1, 0, 2)
    wqi4 = W_q_idx.astype(bf16).reshape(HID, IH, IDIM).transpose(1, 0, 2)
    qsc = q_a_scale.reshape(1, QLORA)
    ksc = kv_a_scale.reshape(1, KVLORA)
    w2 = w_idx.reshape(1, IH)
    inv = 1.0 / (10000.0 ** (np.arange(0, ROPE, 2, dtype=np.float32) / ROPE))
    inv2 = jnp.asarray(np.repeat(inv, 2).reshape(1, ROPE))
    fr = position_ids.reshape(S, 1).astype(jnp.float32) * inv2
    cos_t = jnp.cos(fr)
    sin_t = jnp.sin(fr)
    nq = S // QB
    f32 = jnp.float32

    PB = 512
    q4, qi4, ki, k4, v4 = pl.pallas_call(
        _proj,
        grid=(S // PB,),
        in_specs=[
            pl.BlockSpec((PB, HID), lambda i: (i, 0)),
            _full((HID, QLORA)),
            _full((1, QLORA)),
            _full((H, QLORA, NOPE + ROPE)),
            _full((IH, HID, IDIM)),
            _full((HID, IDIM)),
            _full((HID, KVLORA + ROPE)),
            _full((1, KVLORA)),
            _full((H, KVLORA, NOPE + VDIM)),
            pl.BlockSpec((PB, ROPE), lambda i: (i, 0)),
            pl.BlockSpec((PB, ROPE), lambda i: (i, 0)),
        ],
        out_specs=[
            pl.BlockSpec((H, PB, NOPE + ROPE), lambda i: (0, i, 0)),
            pl.BlockSpec((IH, PB, IDIM), lambda i: (0, i, 0)),
            pl.BlockSpec((PB, IDIM), lambda i: (i, 0)),
            pl.BlockSpec((H, PB, NOPE + ROPE), lambda i: (0, i, 0)),
            pl.BlockSpec((H, PB, VDIM), lambda i: (0, i, 0)),
        ],
        out_shape=[
            jax.ShapeDtypeStruct((H, S, NOPE + ROPE), bf16),
            jax.ShapeDtypeStruct((IH, S, IDIM), bf16),
            jax.ShapeDtypeStruct((S, IDIM), bf16),
            jax.ShapeDtypeStruct((H, S, NOPE + ROPE), bf16),
            jax.ShapeDtypeStruct((H, S, VDIM), bf16),
        ],
    )(hs, wqa_b, qsc, wqb4, wqi4, wki_b, wkva_b, ksc, wkvb4, cos_t, sin_t)

    masks = []
    for ci in range(4):
        width = (ci + 1) * 2 * QB
        tie_iters = max(1, int(np.ceil(np.log2(width))))
        masks.append(pl.pallas_call(
            _make_select(width, ci * 2, tie_iters),
            grid=(1,),
            in_specs=[
                pl.BlockSpec((IH, SB, IDIM), lambda i, c=ci: (0, c, 0)),
                pl.BlockSpec((width, IDIM), lambda i: (0, 0)),
                _full((1, IH)),
            ],
            out_specs=pl.BlockSpec((SB, S), lambda i: (i, 0)),
            out_shape=jax.ShapeDtypeStruct((SB, S), jnp.int8),
        )(qi4, ki, w2))
    mask = jnp.concatenate(masks, axis=0)

    att_parts = []
    for ci in range(4):
        width = (ci + 1) * 2 * QB
        att_parts.append(pl.pallas_call(
            _make_attn(width),
            grid=(H // 4,),
            in_specs=[
                pl.BlockSpec((4, SB, NOPE + ROPE),
                             lambda h, c=ci: (h, c, 0)),
                pl.BlockSpec((4, width, NOPE + ROPE), lambda h: (h, 0, 0)),
                pl.BlockSpec((4, width, VDIM), lambda h: (h, 0, 0)),
                pl.BlockSpec((SB, width), lambda h, c=ci: (c, 0)),
            ],
            out_specs=pl.BlockSpec((SB, 4 * VDIM), lambda h: (0, h)),
            out_shape=jax.ShapeDtypeStruct((SB, H * VDIM), bf16),
        )(q4, k4, v4, mask))
    att2 = jnp.concatenate(att_parts, axis=0)

    out = pl.pallas_call(
        _outproj,
        grid=(nq,),
        in_specs=[
            pl.BlockSpec((QB, H * VDIM), lambda i: (i, 0)),
            _full((H * VDIM, HID)),
        ],
        out_specs=pl.BlockSpec((QB, HID), lambda i: (i, 0)),
        out_shape=jax.ShapeDtypeStruct((S, HID), f32),
    )(att2, wo_b)

    return out.reshape(1, S, HID)


# speedup vs baseline: 16.5970x; 1.0051x over previous
import numpy as np
import jax
import jax.numpy as jnp
from jax.experimental import pallas as pl
from jax.experimental.pallas import tpu as pltpu

S, HID = 2048, 2048
H, NOPE, ROPE, VDIM = 16, 128, 64, 128
QLORA, KVLORA = 1024, 512
IH, IDIM, TOPK = 4, 64, 512
EPS = 1e-5
SCALE = 1.0 / float(np.sqrt(NOPE + ROPE))
QB = 256
NEG = -1e30


def _pairswap(x):
    even = jax.lax.broadcasted_iota(jnp.int32, x.shape, 1) % 2 == 0
    n = x.shape[1]
    return jnp.where(even, -pltpu.roll(x, n - 1, 1), pltpu.roll(x, 1, 1))


def _f32dot(a, b):
    return jnp.dot(a, b, preferred_element_type=jnp.float32)


def _f32dot_nt(a, b):
    return jax.lax.dot_general(
        a, b, (((1,), (1,)), ((), ())), preferred_element_type=jnp.float32)


def _proj(hs_ref, wqa_ref, qsc_ref, wqb_ref, wqi_ref, wki_ref,
          wkva_ref, ksc_ref, wkvb_ref, cos_ref, sin_ref,
          q4_ref, qi4_ref, ki_ref, k4_ref, v4_ref):
    xb = hs_ref[...].astype(jnp.bfloat16)
    c, s = cos_ref[...], sin_ref[...]
    qa = _f32dot(xb, wqa_ref[...])
    var = jnp.mean(qa * qa, axis=1, keepdims=True)
    qn = qa * jax.lax.rsqrt(var + EPS) * qsc_ref[...]
    qnb = qn.astype(jnp.bfloat16)
    kva = _f32dot(xb, wkva_ref[...])
    ckv = kva[:, :KVLORA]
    kr = kva[:, KVLORA:]
    varc = jnp.mean(ckv * ckv, axis=1, keepdims=True)
    cn = ckv * jax.lax.rsqrt(varc + EPS) * ksc_ref[...]
    cnb = cn.astype(jnp.bfloat16)
    kr = kr * c + _pairswap(kr) * s
    krb = kr.astype(jnp.bfloat16)
    for h in range(H):
        qh = _f32dot(qnb, wqb_ref[h])
        qr = qh[:, NOPE:]
        qr = qr * c + _pairswap(qr) * s
        q4_ref[h, :, :NOPE] = qh[:, :NOPE].astype(jnp.bfloat16)
        q4_ref[h, :, NOPE:] = qr.astype(jnp.bfloat16)
        kvh = _f32dot(cnb, wkvb_ref[h])
        k4_ref[h, :, :NOPE] = kvh[:, :NOPE].astype(jnp.bfloat16)
        k4_ref[h, :, NOPE:] = krb
        v4_ref[h] = kvh[:, NOPE:].astype(jnp.bfloat16)
    for g in range(IH):
        qi4_ref[g] = _f32dot(xb, wqi_ref[g]).astype(jnp.bfloat16)
    ki_ref[...] = _f32dot(xb, wki_ref[...]).astype(jnp.bfloat16)


SB = 512


def _make_select(width, c0, tie_iters):
    def _select(qi_ref, ki_ref, w_ref, m_ref):
        ki = ki_ref[...]
        sc = jnp.zeros((SB, width), jnp.float32)
        for g in range(IH):
            r = jnp.maximum(_f32dot_nt(qi_ref[g], ki), 0.0)
            sc = sc + w_ref[0, g] * r.astype(jnp.bfloat16).astype(jnp.float32)
        col = jax.lax.broadcasted_iota(jnp.int32, (SB, width), 1)
        row = c0 * QB + jax.lax.broadcasted_iota(
            jnp.int32, (SB, width), 0)
        causal = col <= row
        key = jnp.where(causal, jax.lax.bitcast_convert_type(sc, jnp.int32),
                        jnp.int32(-1))
        lo = jnp.full((SB, 1), -1, jnp.int32)
        hi = jnp.max(key, axis=1, keepdims=True) + 1
        for _ in range(32):
            mid = lo + jax.lax.shift_right_logical(hi - lo, 1)
            cnt = jnp.sum((key >= mid).astype(jnp.int32), axis=1,
                          keepdims=True)
            ok = cnt >= TOPK
            lo = jnp.where(ok, mid, lo)
            hi = jnp.where(ok, hi, mid)
        t = lo
        cgt = jnp.sum((key > t).astype(jnp.int32), axis=1, keepdims=True)
        need = TOPK - cgt
        eq = key == t
        tlo = jnp.zeros((SB, 1), jnp.int32)
        thi = jnp.full((SB, 1), width, jnp.int32)
        for _ in range(tie_iters):
            mid = tlo + jax.lax.shift_right_logical(thi - tlo, 1)
            cnt = jnp.sum((eq & (col < mid)).astype(jnp.int32),
                          axis=1, keepdims=True)
            ok = cnt >= need
            thi = jnp.where(ok, mid, thi)
            tlo = jnp.where(ok, tlo, mid)
        sel = ((key > t) | (eq & (col < thi))) & causal
        m_ref[:, :width] = sel.astype(jnp.int8)
        if width < S:
            m_ref[:, width:] = jnp.zeros((SB, S - width), jnp.int8)
    return _select


def _make_attn(width):
    def _attn(q_ref, k_ref, v_ref, m_ref, o_ref):
        m = m_ref[...] != 0
        for hh in range(8):
            sc = _f32dot_nt(q_ref[hh], k_ref[hh])
            sc = jnp.where(m, sc * SCALE, NEG)
            mx = jnp.max(sc, axis=1, keepdims=True)
            p = jnp.exp(sc - mx)
            den = jnp.sum(p, axis=1, keepdims=True)
            pb = (p / den).astype(jnp.bfloat16)
            o_ref[:, hh * VDIM:(hh + 1) * VDIM] = (
                _f32dot(pb, v_ref[hh]).astype(jnp.bfloat16))
    return _attn


def _outproj(a_ref, wo_ref, o_ref):
    o_ref[...] = _f32dot(a_ref[...], wo_ref[...])


def _full(shape):
    n = len(shape)
    return pl.BlockSpec(shape, lambda *_: (0,) * n)


@jax.jit
def kernel(hidden_states, position_ids, W_qa, q_a_scale, W_qb, W_kva,
           kv_a_scale, W_kvb, W_o, W_q_idx, W_k_idx, w_idx):
    hs = hidden_states.reshape(S, HID)
    bf16 = jnp.bfloat16
    wqa_b = W_qa.astype(bf16)
    wkva_b = W_kva.astype(bf16)
    wki_b = W_k_idx.astype(bf16)
    wo_b = W_o.astype(bf16)
    wqb4 = W_qb.astype(bf16).reshape(QLORA, H, NOPE + ROPE).transpose(1, 0, 2)
    wkvb4 = W_kvb.astype(bf16).reshape(KVLORA, H, NOPE + VDIM).transpose(1, 0, 2)
    wqi4 = W_q_idx.astype(bf16).reshape(HID, IH, IDIM).transpose(1, 0, 2)
    qsc = q_a_scale.reshape(1, QLORA)
    ksc = kv_a_scale.reshape(1, KVLORA)
    w2 = w_idx.reshape(1, IH)
    inv = 1.0 / (10000.0 ** (np.arange(0, ROPE, 2, dtype=np.float32) / ROPE))
    inv2 = jnp.asarray(np.repeat(inv, 2).reshape(1, ROPE))
    fr = position_ids.reshape(S, 1).astype(jnp.float32) * inv2
    cos_t = jnp.cos(fr)
    sin_t = jnp.sin(fr)
    nq = S // QB
    f32 = jnp.float32

    PB = 512
    q4, qi4, ki, k4, v4 = pl.pallas_call(
        _proj,
        grid=(S // PB,),
        in_specs=[
            pl.BlockSpec((PB, HID), lambda i: (i, 0)),
            _full((HID, QLORA)),
            _full((1, QLORA)),
            _full((H, QLORA, NOPE + ROPE)),
            _full((IH, HID, IDIM)),
            _full((HID, IDIM)),
            _full((HID, KVLORA + ROPE)),
            _full((1, KVLORA)),
            _full((H, KVLORA, NOPE + VDIM)),
            pl.BlockSpec((PB, ROPE), lambda i: (i, 0)),
            pl.BlockSpec((PB, ROPE), lambda i: (i, 0)),
        ],
        out_specs=[
            pl.BlockSpec((H, PB, NOPE + ROPE), lambda i: (0, i, 0)),
            pl.BlockSpec((IH, PB, IDIM), lambda i: (0, i, 0)),
            pl.BlockSpec((PB, IDIM), lambda i: (i, 0)),
            pl.BlockSpec((H, PB, NOPE + ROPE), lambda i: (0, i, 0)),
            pl.BlockSpec((H, PB, VDIM), lambda i: (0, i, 0)),
        ],
        out_shape=[
            jax.ShapeDtypeStruct((H, S, NOPE + ROPE), bf16),
            jax.ShapeDtypeStruct((IH, S, IDIM), bf16),
            jax.ShapeDtypeStruct((S, IDIM), bf16),
            jax.ShapeDtypeStruct((H, S, NOPE + ROPE), bf16),
            jax.ShapeDtypeStruct((H, S, VDIM), bf16),
        ],
    )(hs, wqa_b, qsc, wqb4, wqi4, wki_b, wkva_b, ksc, wkvb4, cos_t, sin_t)

    masks = []
    for ci in range(4):
        width = (ci + 1) * 2 * QB
        tie_iters = max(1, int(np.ceil(np.log2(width))))
        masks.append(pl.pallas_call(
            _make_select(width, ci * 2, tie_iters),
            grid=(1,),
            in_specs=[
                pl.BlockSpec((IH, SB, IDIM), lambda i, c=ci: (0, c, 0)),
                pl.BlockSpec((width, IDIM), lambda i: (0, 0)),
                _full((1, IH)),
            ],
            out_specs=pl.BlockSpec((SB, S), lambda i: (i, 0)),
            out_shape=jax.ShapeDtypeStruct((SB, S), jnp.int8),
        )(qi4, ki, w2))
    mask = jnp.concatenate(masks, axis=0)

    att_parts = []
    for ci in range(4):
        width = (ci + 1) * 2 * QB
        att_parts.append(pl.pallas_call(
            _make_attn(width),
            grid=(H // 8,),
            in_specs=[
                pl.BlockSpec((8, SB, NOPE + ROPE),
                             lambda h, c=ci: (h, c, 0)),
                pl.BlockSpec((8, width, NOPE + ROPE), lambda h: (h, 0, 0)),
                pl.BlockSpec((8, width, VDIM), lambda h: (h, 0, 0)),
                pl.BlockSpec((SB, width), lambda h, c=ci: (c, 0)),
            ],
            out_specs=pl.BlockSpec((SB, 8 * VDIM), lambda h: (0, h)),
            out_shape=jax.ShapeDtypeStruct((SB, H * VDIM), bf16),
        )(q4, k4, v4, mask))
    att2 = jnp.concatenate(att_parts, axis=0)

    out = pl.pallas_call(
        _outproj,
        grid=(nq,),
        in_specs=[
            pl.BlockSpec((QB, H * VDIM), lambda i: (i, 0)),
            _full((H * VDIM, HID)),
        ],
        out_specs=pl.BlockSpec((QB, HID), lambda i: (i, 0)),
        out_shape=jax.ShapeDtypeStruct((S, HID), f32),
    )(att2, wo_b)

    return out.reshape(1, S, HID)
